# baseline (speedup 1.0000x reference)
import math

import jax
import jax.numpy as jnp
from jax.experimental import pallas as pl
from jax.experimental.pallas import tpu as pltpu

B, K, MEM, EDGE, TIME, OUT, H = 10000, 32, 128, 16, 16, 128, 2
DH = OUT // H
BB = 200
BK = BB * K
PACK = MEM // EDGE
BK8 = BK // PACK


def _tgn_block(nm_ref, nb_ref, ef_ref, tf_ref,
               wkm_ref, wke_ref, wkt_ref,
               wvm_ref, wve_ref, wvt_ref,
               wq_ref, bq_ref, wc1_ref, wof_ref, bc_ref, mh_ref,
               out_ref):
    f32 = jnp.float32
    bf16 = jnp.bfloat16
    nb = nb_ref[...].astype(bf16)
    ef = ef_ref[...].astype(bf16)
    tf = tf_ref[...].astype(bf16)
    nm = nm_ref[...]
    nmh = nm.astype(bf16)

    kp = (jnp.dot(nb, wkm_ref[...], preferred_element_type=f32)
          + jnp.dot(ef, wke_ref[...], preferred_element_type=f32).reshape(BK, OUT)
          + jnp.dot(tf, wkt_ref[...], preferred_element_type=f32).reshape(BK, OUT))
    vp = (jnp.dot(nb, wvm_ref[...], preferred_element_type=f32)
          + jnp.dot(ef, wve_ref[...], preferred_element_type=f32).reshape(BK, OUT)
          + jnp.dot(tf, wvt_ref[...], preferred_element_type=f32).reshape(BK, OUT))
    qp = jnp.dot(nmh, wq_ref[...], preferred_element_type=f32) + bq_ref[...]

    prod = (kp.reshape(BB, K, OUT) * qp.reshape(BB, 1, OUT)).reshape(BK, OUT)
    s = jnp.dot(prod.astype(bf16), mh_ref[...], preferred_element_type=f32)
    e3 = jnp.exp(s).reshape(BB, K, OUT)
    vp3 = vp.reshape(BB, K, OUT)
    ctx_un = jnp.sum(e3 * vp3, axis=1)
    denom = jnp.sum(e3, axis=1)
    ctx = ctx_un / denom

    z = (jnp.dot(nmh, wc1_ref[...], preferred_element_type=f32)
         + jnp.dot(ctx.astype(bf16), wof_ref[...], preferred_element_type=f32)
         + bc_ref[...])
    out_ref[...] = jnp.maximum(z, 0.0)


@jax.jit
def kernel(node_mems, neigh_mems, neigh_edge_feats, neigh_dt_enc,
           W_key, b_key, W_val, b_val, W_query, b_query,
           W_q, b_q, W_k, b_k, W_v, b_v, W_o, b_o, W_comb, b_comb):
    scale = 1.0 / math.sqrt(DH)
    Wk_f = (W_k @ W_key) * scale
    Wv_f = W_v @ W_val
    bv_f = W_v @ b_val + b_v
    Wq_f = W_q @ W_query
    bq_f = W_q @ b_query + b_q
    Wc1 = W_comb[:, :MEM]
    Wc2 = W_comb[:, MEM:]
    Wo_f = Wc2 @ W_o
    bc_f = b_comb + Wc2 @ b_o + Wo_f @ bv_f

    bf16 = jnp.bfloat16
    eye = jnp.eye(PACK, dtype=jnp.float32)
    wkm = Wk_f[:, :MEM].T.astype(bf16)
    wke = jnp.kron(eye, Wk_f[:, MEM:MEM + EDGE].T).astype(bf16)
    wkt = jnp.kron(eye, Wk_f[:, MEM + EDGE:].T).astype(bf16)
    wvm = Wv_f[:, :MEM].T.astype(bf16)
    wve = jnp.kron(eye, Wv_f[:, MEM:MEM + EDGE].T).astype(bf16)
    wvt = jnp.kron(eye, Wv_f[:, MEM + EDGE:].T).astype(bf16)
    wq = Wq_f.T.astype(bf16)
    wc1 = Wc1.T.astype(bf16)
    wof = Wo_f.T.astype(bf16)

    lane = jnp.arange(OUT)
    mh = (lane[:, None] // DH == lane[None, :] // DH).astype(bf16)

    def row2d(v):
        return v.reshape(1, OUT)

    grid = (B // BB,)
    full = lambda shape: pl.BlockSpec(shape, lambda i: tuple(0 for _ in shape))
    out = pl.pallas_call(
        _tgn_block,
        grid=grid,
        in_specs=[
            pl.BlockSpec((BB, MEM), lambda i: (i, 0)),
            pl.BlockSpec((BK, MEM), lambda i: (i, 0)),
            pl.BlockSpec((BK8, MEM), lambda i: (i, 0)),
            pl.BlockSpec((BK8, MEM), lambda i: (i, 0)),
            full((MEM, OUT)), full((MEM, PACK * OUT)), full((MEM, PACK * OUT)),
            full((MEM, OUT)), full((MEM, PACK * OUT)), full((MEM, PACK * OUT)),
            full((MEM, OUT)), full((1, OUT)),
            full((MEM, OUT)), full((OUT, OUT)), full((1, OUT)),
            full((OUT, OUT)),
        ],
        out_specs=pl.BlockSpec((BB, OUT), lambda i: (i, 0)),
        out_shape=jax.ShapeDtypeStruct((B, OUT), jnp.float32),
        compiler_params=pltpu.CompilerParams(
            dimension_semantics=("arbitrary",),
        ),
    )(node_mems,
      neigh_mems.reshape(B * K, MEM),
      neigh_edge_feats.reshape(B * K * EDGE // MEM, MEM),
      neigh_dt_enc.reshape(B * K * TIME // MEM, MEM),
      wkm, wke, wkt,
      wvm, wve, wvt,
      wq, row2d(bq_f),
      wc1, wof, row2d(bc_f), mh)
    return out


# speedup vs baseline: 2.1308x; 2.1308x over previous
import jax
import jax.numpy as jnp
from jax import lax
from jax.experimental import pallas as pl
from jax.experimental.pallas import tpu as pltpu

B, K, MEM, EDGE, TIME, OUT, H = 10000, 32, 128, 16, 16, 128, 2
DH = OUT // H
IN = MEM + EDGE + TIME
BB = 400
BK = BB * K


def _dg(a, b, a_dim, b_dim):
    return lax.dot_general(a, b, (((a_dim,), (b_dim,)), ((), ())),
                           preferred_element_type=jnp.float32)


def _tgn_block(nm_ref, nb_ref, ef_ref, tf_ref,
               wkey_ref, wval_ref, wquery_ref, wq_ref, wk_ref, wv_ref,
               wo_ref, wcomb_ref,
               bval_ref, bvp_ref, bquery_ref, bq_ref, bo_ref, bcomb_ref,
               out_ref,
               wkvm_s, wkve_s, wkvt_s, wq_s, bq_s, wc1_s, wof_s, bc_s, mh_s):
    f32 = jnp.float32
    bf16 = jnp.bfloat16

    @pl.when(pl.program_id(0) == 0)
    def _fuse_weights():
        wkey = wkey_ref[...]
        wval = wval_ref[...]
        wk = wk_ref[...]
        wv = wv_ref[...]
        for s_ref, lo, hi in ((wkvm_s, 0, MEM),
                              (wkve_s, MEM, MEM + EDGE),
                              (wkvt_s, MEM + EDGE, IN)):
            kpart = _dg(wkey[:, lo:hi], wk, 0, 1)
            vpart = _dg(wval[:, lo:hi], wv, 0, 1)
            s_ref[...] = jnp.concatenate([kpart, vpart], axis=1).astype(bf16)
        wq_s[...] = _dg(wquery_ref[...], wq_ref[...], 0, 1).astype(bf16)
        bq_s[...] = _dg(bquery_ref[...], wq_ref[...], 1, 1) + bq_ref[...]
        wc1 = wcomb_ref[...][:, :MEM]
        wc2 = wcomb_ref[...][:, MEM:]
        wc1_s[...] = jnp.transpose(wc1).astype(bf16)
        wof = _dg(wo_ref[...], wc2, 0, 1)
        wof_s[...] = wof.astype(bf16)
        bv_f = _dg(bval_ref[...], wv, 1, 1) + bvp_ref[...]
        bc_s[...] = (bcomb_ref[...]
                     + _dg(bo_ref[...], wc2, 1, 1)
                     + _dg(bv_f, wof, 1, 0))
        ji = lax.broadcasted_iota(jnp.int32, (OUT, OUT), 0) // DH
        li = lax.broadcasted_iota(jnp.int32, (OUT, OUT), 1) // DH
        mh_s[...] = jnp.where(ji == li, 0.125, 0.0).astype(bf16)

    nb = nb_ref[...].astype(bf16)
    ef = ef_ref[...]
    tf = tf_ref[...]
    nm = nm_ref[...]
    nmh = nm.astype(bf16)

    kv = (jnp.dot(nb, wkvm_s[...], preferred_element_type=f32)
          + jnp.dot(ef, wkve_s[...], preferred_element_type=f32)
          + jnp.dot(tf, wkvt_s[...], preferred_element_type=f32))
    kp = kv[:, :OUT]
    vp = kv[:, OUT:]
    qp = jnp.dot(nmh, wq_s[...], preferred_element_type=f32) + bq_s[...]

    prod = (kp.reshape(BB, K, OUT) * qp.reshape(BB, 1, OUT)).reshape(BK, OUT)
    s = jnp.dot(prod.astype(bf16), mh_s[...], preferred_element_type=f32)
    e3 = jnp.exp(s).reshape(BB, K, OUT)
    vp3 = vp.reshape(BB, K, OUT)
    ctx_un = jnp.sum(e3 * vp3, axis=1)
    denom = jnp.sum(e3, axis=1)
    ctx = ctx_un / denom

    z = (jnp.dot(nmh, wc1_s[...], preferred_element_type=f32)
         + jnp.dot(ctx.astype(bf16), wof_s[...], preferred_element_type=f32)
         + bc_s[...])
    out_ref[...] = jnp.maximum(z, 0.0)


@jax.jit
def kernel(node_mems, neigh_mems, neigh_edge_feats, neigh_dt_enc,
           W_key, b_key, W_val, b_val, W_query, b_query,
           W_q, b_q, W_k, b_k, W_v, b_v, W_o, b_o, W_comb, b_comb):
    bf16 = jnp.bfloat16
    ef = neigh_edge_feats.reshape(B * K, EDGE).astype(bf16)
    tf = neigh_dt_enc.reshape(B * K, TIME).astype(bf16)

    def row(v):
        return v.reshape(1, OUT)

    grid = (B // BB,)
    full = lambda shape: pl.BlockSpec(shape, lambda i: tuple(0 for _ in shape))
    out = pl.pallas_call(
        _tgn_block,
        grid=grid,
        in_specs=[
            pl.BlockSpec((BB, MEM), lambda i: (i, 0)),
            pl.BlockSpec((BK, MEM), lambda i: (i, 0)),
            pl.BlockSpec((BK, EDGE), lambda i: (i, 0)),
            pl.BlockSpec((BK, TIME), lambda i: (i, 0)),
            full((OUT, IN)), full((OUT, IN)), full((OUT, MEM)),
            full((OUT, OUT)), full((OUT, OUT)), full((OUT, OUT)),
            full((OUT, OUT)), full((OUT, MEM + OUT)),
            full((1, OUT)), full((1, OUT)), full((1, OUT)),
            full((1, OUT)), full((1, OUT)), full((1, OUT)),
        ],
        out_specs=pl.BlockSpec((BB, OUT), lambda i: (i, 0)),
        out_shape=jax.ShapeDtypeStruct((B, OUT), jnp.float32),
        scratch_shapes=[
            pltpu.VMEM((MEM, 2 * OUT), bf16),
            pltpu.VMEM((EDGE, 2 * OUT), bf16),
            pltpu.VMEM((TIME, 2 * OUT), bf16),
            pltpu.VMEM((MEM, OUT), bf16),
            pltpu.VMEM((1, OUT), jnp.float32),
            pltpu.VMEM((MEM, OUT), bf16),
            pltpu.VMEM((OUT, OUT), bf16),
            pltpu.VMEM((1, OUT), jnp.float32),
            pltpu.VMEM((OUT, OUT), bf16),
        ],
        compiler_params=pltpu.CompilerParams(
            dimension_semantics=("arbitrary",),
        ),
    )(node_mems,
      neigh_mems.reshape(B * K, MEM),
      ef, tf,
      W_key, W_val, W_query, W_q, W_k, W_v, W_o, W_comb,
      row(b_val), row(b_v), row(b_query), row(b_q), row(b_o), row(b_comb))
    return out


# speedup vs baseline: 2.1444x; 1.0064x over previous
import math

import jax
import jax.numpy as jnp
from jax.experimental import pallas as pl
from jax.experimental.pallas import tpu as pltpu

B, K, MEM, EDGE, TIME, OUT, H = 10000, 32, 128, 16, 16, 128, 2
DH = OUT // H
BB = 400
BK = BB * K
PACK = MEM // EDGE
BK8 = BK // PACK


def _tgn_block(nm_ref, nb_ref, ef_ref, tf_ref,
               wkm_ref, wke_ref, wkt_ref,
               wq_ref, bq_ref, wc1_ref, wof_ref, bc_ref, mh_ref,
               out_ref):
    f32 = jnp.float32
    bf16 = jnp.bfloat16
    nb = nb_ref[...].astype(bf16)
    ef = ef_ref[...]
    tf = tf_ref[...]
    nm = nm_ref[...]
    nmh = nm.astype(bf16)

    kv = (jnp.dot(nb, wkm_ref[...], preferred_element_type=f32)
          + jnp.dot(ef, wke_ref[...], preferred_element_type=f32)
          + jnp.dot(tf, wkt_ref[...], preferred_element_type=f32))
    kp = kv[:, :OUT]
    vp = kv[:, OUT:]
    qp = jnp.dot(nmh, wq_ref[...], preferred_element_type=f32) + bq_ref[...]

    prod = (kp.reshape(BB, K, OUT) * qp.reshape(BB, 1, OUT)).reshape(BK, OUT)
    s = jnp.dot(prod.astype(bf16), mh_ref[...], preferred_element_type=f32)
    e3 = jnp.exp(s).reshape(BB, K, OUT)
    vp3 = vp.reshape(BB, K, OUT)
    ctx_un = jnp.sum(e3 * vp3, axis=1)
    denom = jnp.sum(e3, axis=1)
    ctx = ctx_un / denom

    z = (jnp.dot(nmh, wc1_ref[...], preferred_element_type=f32)
         + jnp.dot(ctx.astype(bf16), wof_ref[...], preferred_element_type=f32)
         + bc_ref[...])
    out_ref[...] = jnp.maximum(z, 0.0)


@jax.jit
def kernel(node_mems, neigh_mems, neigh_edge_feats, neigh_dt_enc,
           W_key, b_key, W_val, b_val, W_query, b_query,
           W_q, b_q, W_k, b_k, W_v, b_v, W_o, b_o, W_comb, b_comb):
    scale = 1.0 / math.sqrt(DH)
    Wk_f = (W_k @ W_key) * scale
    Wv_f = W_v @ W_val
    bv_f = W_v @ b_val + b_v
    Wq_f = W_q @ W_query
    bq_f = W_q @ b_query + b_q
    Wc1 = W_comb[:, :MEM]
    Wc2 = W_comb[:, MEM:]
    Wo_f = Wc2 @ W_o
    bc_f = b_comb + Wc2 @ b_o + Wo_f @ bv_f

    bf16 = jnp.bfloat16
    Wkv = jnp.concatenate([Wk_f, Wv_f], axis=0)
    wkm = Wkv[:, :MEM].T.astype(bf16)
    wke = Wkv[:, MEM:MEM + EDGE].T.astype(bf16)
    wkt = Wkv[:, MEM + EDGE:].T.astype(bf16)
    wq = Wq_f.T.astype(bf16)
    wc1 = Wc1.T.astype(bf16)
    wof = Wo_f.T.astype(bf16)

    lane = jnp.arange(OUT)
    mh = (lane[:, None] // DH == lane[None, :] // DH).astype(bf16)

    def row2d(v):
        return v.reshape(1, OUT)

    ef = neigh_edge_feats.reshape(B * K, EDGE).astype(bf16)
    tf = neigh_dt_enc.reshape(B * K, TIME).astype(bf16)

    grid = (B // BB,)
    full = lambda shape: pl.BlockSpec(shape, lambda i: tuple(0 for _ in shape))
    out = pl.pallas_call(
        _tgn_block,
        grid=grid,
        in_specs=[
            pl.BlockSpec((BB, MEM), lambda i: (i, 0)),
            pl.BlockSpec((BK, MEM), lambda i: (i, 0)),
            pl.BlockSpec((BK, EDGE), lambda i: (i, 0)),
            pl.BlockSpec((BK, TIME), lambda i: (i, 0)),
            full((MEM, 2 * OUT)), full((EDGE, 2 * OUT)), full((TIME, 2 * OUT)),
            full((MEM, OUT)), full((1, OUT)),
            full((MEM, OUT)), full((OUT, OUT)), full((1, OUT)),
            full((OUT, OUT)),
        ],
        out_specs=pl.BlockSpec((BB, OUT), lambda i: (i, 0)),
        out_shape=jax.ShapeDtypeStruct((B, OUT), jnp.float32),
        compiler_params=pltpu.CompilerParams(
            dimension_semantics=("parallel",),
        ),
    )(node_mems,
      neigh_mems.reshape(B * K, MEM),
      ef, tf,
      wkm, wke, wkt,
      wq, row2d(bq_f),
      wc1, wof, row2d(bc_f), mh)
    return out
